# baseline (device time: 32287 ns/iter reference)
import jax
import jax.numpy as jnp
from jax import lax
from jax.experimental import pallas as pl
from jax.experimental.pallas import tpu as pltpu

N_DEV = 32
C = 16


def kernel(x, router_W, route_idx, expert_W):
    n, d = x.shape
    n_exp = router_W.shape[1]
    e_local = expert_W.shape[0]
    h = expert_W.shape[2]

    def body(x_ref, rw_ref, idx_ref, ew_ref, out_ref,
             stage_ref, recv_ref, gather_ref, send_sems, recv_sems):
        my = lax.axis_index("i")

        barrier = pltpu.get_barrier_semaphore()
        for r in range(1, N_DEV):
            pl.semaphore_signal(barrier, inc=1,
                                device_id=(lax.rem(my + r, N_DEV),),
                                device_id_type=pl.DeviceIdType.MESH)
        pl.semaphore_wait(barrier, N_DEV - 1)

        xf = x_ref[:, :]
        scores = jnp.dot(xf, rw_ref[:, :],
                         preferred_element_type=jnp.float32)
        smax = jnp.max(scores, axis=1, keepdims=True)
        p = jnp.exp(scores - smax)
        probs = p / jnp.sum(p, axis=1, keepdims=True)

        idx0 = idx_ref[:, 0:1]
        idx1 = idx_ref[:, 1:2]
        cols = lax.broadcasted_iota(jnp.int32, (n, n_exp), 1)
        g0 = jnp.sum(jnp.where(cols == idx0, probs, 0.0), axis=1,
                     keepdims=True)
        g1 = jnp.sum(jnp.where(cols == idx1, probs, 0.0), axis=1,
                     keepdims=True)
        gs = g0 + g1

        acc = jnp.zeros((n, h), jnp.float32)
        for j in range(e_local):
            e = my * e_local + j
            p_e = jnp.sum(jnp.where(cols == e, probs, 0.0), axis=1,
                          keepdims=True)
            mask = jnp.logical_or(idx0 == e, idx1 == e)
            g_e = jnp.where(mask, p_e / gs, 0.0)
            xg = (xf * g_e).astype(jnp.bfloat16)
            acc = acc + jnp.dot(xg, ew_ref[j].astype(jnp.bfloat16),
                                preferred_element_type=jnp.float32)
        stage_ref[:, :] = acc.astype(jnp.bfloat16)
        out_ref[:, :] = acc

        rs = []
        for r in range(1, N_DEV):
            t = lax.rem(my + r, N_DEV)
            rdma = pltpu.make_async_remote_copy(
                src_ref=stage_ref.at[pl.ds(t * C, C)],
                dst_ref=recv_ref.at[r - 1],
                send_sem=send_sems.at[r - 1],
                recv_sem=recv_sems.at[r - 1],
                device_id=(t,),
                device_id_type=pl.DeviceIdType.MESH,
            )
            rdma.start()
            rs.append(rdma)

        for rdma in rs:
            rdma.wait_recv()
        red = (
            out_ref[pl.ds(my * C, C), :]
            + jnp.sum(recv_ref[:, :, :].astype(jnp.float32), axis=0)
        )
        gather_ref[pl.ds(my * C, C), :] = red.astype(jnp.bfloat16)

        ag = []
        for r in range(1, N_DEV):
            t = lax.rem(my + r, N_DEV)
            rdma = pltpu.make_async_remote_copy(
                src_ref=gather_ref.at[pl.ds(my * C, C)],
                dst_ref=gather_ref.at[pl.ds(my * C, C)],
                send_sem=send_sems.at[N_DEV - 1 + r - 1],
                recv_sem=recv_sems.at[N_DEV - 1 + r - 1],
                device_id=(t,),
                device_id_type=pl.DeviceIdType.MESH,
            )
            rdma.start()
            ag.append(rdma)

        for rdma in ag:
            rdma.wait_recv()
        out_ref[:, :] = gather_ref[:, :].astype(jnp.float32)

        for rdma in rs:
            rdma.wait_send()
        for rdma in ag:
            rdma.wait_send()

    return pl.pallas_call(
        body,
        out_shape=jax.ShapeDtypeStruct((n, h), jnp.float32),
        in_specs=[pl.BlockSpec(memory_space=pltpu.VMEM)] * 4,
        out_specs=pl.BlockSpec(memory_space=pltpu.VMEM),
        scratch_shapes=[
            pltpu.VMEM((n, h), jnp.bfloat16),
            pltpu.VMEM((N_DEV - 1, C, h), jnp.bfloat16),
            pltpu.VMEM((n, h), jnp.bfloat16),
            pltpu.SemaphoreType.DMA((2 * (N_DEV - 1),)),
            pltpu.SemaphoreType.DMA((2 * (N_DEV - 1),)),
        ],
        compiler_params=pltpu.CompilerParams(collective_id=0),
    )(x, router_W, route_idx, expert_W)


# device time: 27862 ns/iter; 1.1588x vs baseline; 1.1588x over previous
import jax
import jax.numpy as jnp
from jax import lax
from jax.experimental import pallas as pl
from jax.experimental.pallas import tpu as pltpu

N_DEV = 32
C = 16


def kernel(x, router_W, route_idx, expert_W):
    n, d = x.shape
    n_exp = router_W.shape[1]
    e_local = expert_W.shape[0]
    h = expert_W.shape[2]

    def body(x_ref, rw_ref, idx_ref, ew_ref, out_ref,
             stage_ref, recv_ref, gather_ref, send_sems, recv_sems):
        my = lax.axis_index("i")

        barrier = pltpu.get_barrier_semaphore()
        for r in range(1, N_DEV):
            pl.semaphore_signal(barrier, inc=1,
                                device_id=(lax.rem(my + r, N_DEV),),
                                device_id_type=pl.DeviceIdType.MESH)

        xf = x_ref[:, :]
        scores = jnp.dot(xf, rw_ref[:, :],
                         preferred_element_type=jnp.float32)
        smax = jnp.max(scores, axis=1, keepdims=True)
        p = jnp.exp(scores - smax)

        idx0 = idx_ref[:, 0:1]
        idx1 = idx_ref[:, 1:2]
        cols = lax.broadcasted_iota(jnp.int32, (n, n_exp), 1)
        g0 = jnp.sum(jnp.where(cols == idx0, p, 0.0), axis=1,
                     keepdims=True)
        g1 = jnp.sum(jnp.where(cols == idx1, p, 0.0), axis=1,
                     keepdims=True)
        gs = g0 + g1

        gated = []
        for j in range(e_local):
            e = my * e_local + j
            p_e = jnp.sum(jnp.where(cols == e, p, 0.0), axis=1,
                          keepdims=True)
            mask = jnp.logical_or(idx0 == e, idx1 == e)
            g_e = jnp.where(mask, p_e / gs, 0.0)
            gated.append((xf * g_e).astype(jnp.bfloat16))
        xg = jnp.concatenate(gated, axis=1)
        w = jnp.concatenate([ew_ref[j].astype(jnp.bfloat16)
                             for j in range(e_local)], axis=0)
        acc = jnp.dot(xg, w, preferred_element_type=jnp.float32)
        stage_ref[:, :] = acc.astype(jnp.bfloat16)

        pl.semaphore_wait(barrier, N_DEV - 1)

        rs = []
        for r in range(1, N_DEV):
            t = lax.rem(my + r, N_DEV)
            rdma = pltpu.make_async_remote_copy(
                src_ref=stage_ref.at[pl.ds(t * C, C)],
                dst_ref=recv_ref.at[r - 1],
                send_sem=send_sems.at[r - 1],
                recv_sem=recv_sems.at[r - 1],
                device_id=(t,),
                device_id_type=pl.DeviceIdType.MESH,
            )
            rdma.start()
            rs.append(rdma)

        for rdma in rs:
            rdma.wait_recv()
        red = (
            stage_ref[pl.ds(my * C, C), :].astype(jnp.float32)
            + jnp.sum(recv_ref[:, :, :].astype(jnp.float32), axis=0)
        )
        gather_ref[pl.ds(my * C, C), :] = red.astype(jnp.bfloat16)

        ag = []
        for r in range(1, N_DEV):
            t = lax.rem(my + r, N_DEV)
            rdma = pltpu.make_async_remote_copy(
                src_ref=gather_ref.at[pl.ds(my * C, C)],
                dst_ref=gather_ref.at[pl.ds(my * C, C)],
                send_sem=send_sems.at[N_DEV - 1 + r - 1],
                recv_sem=recv_sems.at[N_DEV - 1 + r - 1],
                device_id=(t,),
                device_id_type=pl.DeviceIdType.MESH,
            )
            rdma.start()
            ag.append(rdma)

        for rdma in ag:
            rdma.wait_recv()
        out_ref[:, :] = gather_ref[:, :].astype(jnp.float32)

        for rdma in rs:
            rdma.wait_send()
        for rdma in ag:
            rdma.wait_send()

    return pl.pallas_call(
        body,
        out_shape=jax.ShapeDtypeStruct((n, h), jnp.float32),
        in_specs=[pl.BlockSpec(memory_space=pltpu.VMEM)] * 4,
        out_specs=pl.BlockSpec(memory_space=pltpu.VMEM),
        scratch_shapes=[
            pltpu.VMEM((n, h), jnp.bfloat16),
            pltpu.VMEM((N_DEV - 1, C, h), jnp.bfloat16),
            pltpu.VMEM((n, h), jnp.bfloat16),
            pltpu.SemaphoreType.DMA((2 * (N_DEV - 1),)),
            pltpu.SemaphoreType.DMA((2 * (N_DEV - 1),)),
        ],
        compiler_params=pltpu.CompilerParams(collective_id=0),
    )(x, router_W, route_idx, expert_W)


# device time: 27236 ns/iter; 1.1855x vs baseline; 1.0230x over previous
import jax
import jax.numpy as jnp
from jax import lax
from jax.experimental import pallas as pl
from jax.experimental.pallas import tpu as pltpu

N_DEV = 32
C = 16
NP = N_DEV - 1


def kernel(x, router_W, route_idx, expert_W):
    n, d = x.shape
    n_exp = router_W.shape[1]
    e_local = expert_W.shape[0]
    h = expert_W.shape[2]
    hh = h // 2

    def body(x_ref, rw_ref, idx_ref, ew_ref, out_ref,
             stage_a, stage_b, recv_a, recv_b, gather_a, gather_b,
             send_sems, recv_sems):
        my = lax.axis_index("i")

        barrier = pltpu.get_barrier_semaphore()
        for r in range(1, N_DEV):
            pl.semaphore_signal(barrier, inc=1,
                                device_id=(lax.rem(my + r, N_DEV),),
                                device_id_type=pl.DeviceIdType.MESH)

        xf = x_ref[:, :]
        scores = jnp.dot(xf, rw_ref[:, :],
                         preferred_element_type=jnp.float32)
        smax = jnp.max(scores, axis=1, keepdims=True)
        p = jnp.exp(scores - smax)

        idx0 = idx_ref[:, 0:1]
        idx1 = idx_ref[:, 1:2]
        cols = lax.broadcasted_iota(jnp.int32, (n, n_exp), 1)
        g0 = jnp.sum(jnp.where(cols == idx0, p, 0.0), axis=1,
                     keepdims=True)
        g1 = jnp.sum(jnp.where(cols == idx1, p, 0.0), axis=1,
                     keepdims=True)
        gs = g0 + g1

        gated = []
        for j in range(e_local):
            e = my * e_local + j
            p_e = jnp.sum(jnp.where(cols == e, p, 0.0), axis=1,
                          keepdims=True)
            mask = jnp.logical_or(idx0 == e, idx1 == e)
            g_e = jnp.where(mask, p_e / gs, 0.0)
            gated.append((xf * g_e).astype(jnp.bfloat16))
        xg = jnp.concatenate(gated, axis=1)
        w = jnp.concatenate([ew_ref[j].astype(jnp.bfloat16)
                             for j in range(e_local)], axis=0)
        acc = jnp.dot(xg, w, preferred_element_type=jnp.float32)
        stage_a[:, :] = acc[:, :hh].astype(jnp.bfloat16)
        stage_b[:, :] = acc[:, hh:].astype(jnp.bfloat16)

        pl.semaphore_wait(barrier, NP)

        def a2a(src_ref, dst_ref, sem_base, sends):
            for r in range(1, N_DEV):
                t = lax.rem(my + r, N_DEV)
                rdma = pltpu.make_async_remote_copy(
                    src_ref=src_ref.at[pl.ds(t * C, C)],
                    dst_ref=(dst_ref.at[r - 1] if dst_ref.ndim == 3
                             else dst_ref.at[pl.ds(my * C, C)]),
                    send_sem=send_sems.at[sem_base + r - 1],
                    recv_sem=recv_sems.at[sem_base + r - 1],
                    device_id=(t,),
                    device_id_type=pl.DeviceIdType.MESH,
                )
                rdma.start()
                sends.append(rdma)

        rs_a, rs_b = [], []
        a2a(stage_a, recv_a, 0 * NP, rs_a)
        a2a(stage_b, recv_b, 1 * NP, rs_b)

        ag_a, ag_b = [], []
        for rdma in rs_a:
            rdma.wait_recv()
        red_a = (
            stage_a[pl.ds(my * C, C), :].astype(jnp.float32)
            + jnp.sum(recv_a[:, :, :].astype(jnp.float32), axis=0)
        )
        gather_a[pl.ds(my * C, C), :] = red_a.astype(jnp.bfloat16)
        a2a(gather_a, gather_a, 2 * NP, ag_a)

        for rdma in rs_b:
            rdma.wait_recv()
        red_b = (
            stage_b[pl.ds(my * C, C), :].astype(jnp.float32)
            + jnp.sum(recv_b[:, :, :].astype(jnp.float32), axis=0)
        )
        gather_b[pl.ds(my * C, C), :] = red_b.astype(jnp.bfloat16)
        a2a(gather_b, gather_b, 3 * NP, ag_b)

        for rdma in ag_a:
            rdma.wait_recv()
        out_ref[:, :hh] = gather_a[:, :].astype(jnp.float32)
        for rdma in ag_b:
            rdma.wait_recv()
        out_ref[:, hh:] = gather_b[:, :].astype(jnp.float32)

        for rdma in rs_a + rs_b + ag_a + ag_b:
            rdma.wait_send()

    return pl.pallas_call(
        body,
        out_shape=jax.ShapeDtypeStruct((n, h), jnp.float32),
        in_specs=[pl.BlockSpec(memory_space=pltpu.VMEM)] * 4,
        out_specs=pl.BlockSpec(memory_space=pltpu.VMEM),
        scratch_shapes=[
            pltpu.VMEM((n, hh), jnp.bfloat16),
            pltpu.VMEM((n, hh), jnp.bfloat16),
            pltpu.VMEM((NP, C, hh), jnp.bfloat16),
            pltpu.VMEM((NP, C, hh), jnp.bfloat16),
            pltpu.VMEM((n, hh), jnp.bfloat16),
            pltpu.VMEM((n, hh), jnp.bfloat16),
            pltpu.SemaphoreType.DMA((4 * NP,)),
            pltpu.SemaphoreType.DMA((4 * NP,)),
        ],
        compiler_params=pltpu.CompilerParams(collective_id=0),
    )(x, router_W, route_idx, expert_W)
